# baseline (speedup 1.0000x reference)
import functools

import jax
import jax.numpy as jnp
from jax import lax
from jax.experimental import pallas as pl
from jax.experimental.pallas import tpu as pltpu
from jax.experimental.pallas import tpu_sc as plsc

L = 128
N = 10000
E = 320000
D1 = 2 * L
D2 = 4 * D1 + 2 * L

EBLK = 1024
E_PAD = 327680
NBLK = 1000

NTILES = 16
EPT = E_PAD // NTILES
NCH = 160
ACC_ROWS = 10240
DUMMY = N
W = 128

_INTERPRET = False


def _leaky(x, slope):
    return jnp.where(x >= 0, x, slope * x)


def _dot_t(a, w):
    return jax.lax.dot_general(a, w, (((1,), (1,)), ((), ())),
                               preferred_element_type=jnp.float32)


GPT = E_PAD // 32
GCH = GPT // 128


@functools.partial(
    pl.kernel,
    out_type=jax.ShapeDtypeStruct((E_PAD, L), jnp.float32),
    mesh=plsc.VectorSubcoreMesh(core_axis_name="c", subcore_axis_name="s"),
    scratch_types=[
        pltpu.VMEM_SHARED((ACC_ROWS, L), jnp.float32),
        pltpu.VMEM((GCH, 128), jnp.int32),
        pltpu.VMEM((128, L), jnp.float32),
        pltpu.VMEM((128, L), jnp.float32),
        pltpu.SemaphoreType.DMA,
        pltpu.SemaphoreType.DMA,
        pltpu.SemaphoreType.DMA,
        pltpu.SemaphoreType.DMA,
    ],
)
def _sc_gather(xt_hbm, tgt_hbm, out_hbm, xts, idx_v, r0b, r1b, g0, g1, w0, w1):
    c = lax.axis_index("c")
    s = lax.axis_index("s")
    w = s * 2 + c
    pltpu.sync_copy(xt_hbm.at[pl.ds(s * 640, 640)], xts.at[pl.ds(s * 640, 640)])
    pltpu.sync_copy(tgt_hbm.at[w], idx_v)
    plsc.subcore_barrier()

    rbs = (r0b, r1b)
    gsems = (g0, g1)
    wsems = (w0, w1)
    base = w * GPT

    def group(g, carry):
        hg = [None] * 8
        hw = [None] * 8
        row0 = base + g * (8 * 128)
        hg[0] = pltpu.async_copy(xts.at[idx_v.at[g * 8]], r0b, g0)
        for k in range(8):
            b = k % 2
            hg[k].wait()
            if k + 1 < 8:
                if k >= 1:
                    hw[k - 1].wait()
                hg[k + 1] = pltpu.async_copy(
                    xts.at[idx_v.at[g * 8 + k + 1]], rbs[1 - b],
                    gsems[1 - b])
            hw[k] = pltpu.async_copy(
                rbs[b], out_hbm.at[pl.ds(row0 + k * 128, 128)], wsems[b])
        hw[6].wait()
        hw[7].wait()
        return carry
    lax.fori_loop(0, GCH // 8, group, 0)


def _edge_body(xt_ref, ea_ref, w1a_ref, b1a_ref, w1b_ref, b1b_ref, msg_ref):
    x = jnp.concatenate([xt_ref[...], ea_ref[...]], axis=1)
    h = _dot_t(x, w1a_ref[...]) + b1a_ref[...]
    h = _leaky(h, 0.1)
    msg = _dot_t(h, w1b_ref[...]) + b1b_ref[...]
    msg_ref[0] = msg[:, :L]
    msg_ref[1] = msg[:, L:]


def _edge_msg(xt_g, edge_attr, W1a, b1a, W1b, b1b):
    nsteps = E_PAD // EBLK
    last = E // EBLK - (1 if E % EBLK == 0 else 0)

    def in_map(j):
        return (jnp.minimum(j, last), 0)

    return pl.pallas_call(
        _edge_body,
        grid=(nsteps,),
        in_specs=[
            pl.BlockSpec((EBLK, L), lambda j: (j, 0)),
            pl.BlockSpec((EBLK, L), in_map),
            pl.BlockSpec((D1, D1), lambda j: (0, 0)),
            pl.BlockSpec((1, D1), lambda j: (0, 0)),
            pl.BlockSpec((D1, D1), lambda j: (0, 0)),
            pl.BlockSpec((1, D1), lambda j: (0, 0)),
        ],
        out_specs=pl.BlockSpec((2, EBLK, L), lambda j: (0, j, 0)),
        out_shape=jax.ShapeDtypeStruct((2, E_PAD, L), jnp.float32),
        interpret=_INTERPRET,
    )(xt_g, edge_attr, W1a, b1a.reshape(1, D1), W1b, b1b.reshape(1, D1))


WSC = 32


WSC = 32


def _sc_pows(mb, pb2, pb3, pb4, nrows):
    @plsc.parallel_loop(0, nrows, unroll=8)
    def pw(r):
        for j in range(WSC // 16):
            v = mb[r, pl.ds(j * 16, 16)]
            v2 = v * v
            pb2[r, pl.ds(j * 16, 16)] = v2
            pb3[r, pl.ds(j * 16, 16)] = v2 * v
            pb4[r, pl.ds(j * 16, 16)] = v2 * v2


@functools.partial(
    pl.kernel,
    out_type=jax.ShapeDtypeStruct((4, 2, ACC_ROWS, L), jnp.float32),
    mesh=plsc.VectorSubcoreMesh(core_axis_name="c", subcore_axis_name="s"),
    compiler_params=pltpu.CompilerParams(use_tc_tiling_on_sc=False),
    scratch_types=[
        pltpu.VMEM_SHARED((4, ACC_ROWS, WSC), jnp.float32),
        pltpu.VMEM((8, 128), jnp.int32),
        pltpu.VMEM((2, 128, WSC), jnp.float32),
        pltpu.VMEM((2, 128, WSC), jnp.float32),
        pltpu.VMEM((2, 128, WSC), jnp.float32),
        pltpu.VMEM((2, 128, WSC), jnp.float32),
        pltpu.VMEM((128, WSC), jnp.float32),
        pltpu.SemaphoreType.DMA,
        pltpu.SemaphoreType.DMA,
        pltpu.SemaphoreType.DMA,
        pltpu.SemaphoreType.DMA,
    ],
)
def _sc_scatter(msg_hbm, idx_hbm, out_hbm, acc, idx_v, mb, pb2, pb3, pb4,
                zb, l0, l1, s0, s1):
    c = lax.axis_index("c")
    t = lax.axis_index("s")
    base = t * EPT
    lsems = (l0, l1)
    ssems = (s0, s1)

    zv = jnp.zeros((16,), jnp.float32)

    def _z(i, carry):
        for j in range(WSC // 16):
            zb[i, pl.ds(j * 16, 16)] = zv
        return carry
    lax.fori_loop(0, 128, _z, 0)

    for cg in range(4):
        coff = cg * WSC

        for m in range(4):
            for z in range(5):
                pltpu.sync_copy(zb,
                                acc.at[m, pl.ds(t * 640 + z * 128, 128)])
        plsc.subcore_barrier()

        def group(g, carry):
            pltpu.sync_copy(idx_hbm.at[t, pl.ds(g * 8, 8)], idx_v)
            g0 = base + g * (8 * 128)
            hl = [None] * 8
            hs = [None] * 8
            hl[0] = pltpu.async_copy(
                msg_hbm.at[c, pl.ds(g0, 128), pl.ds(coff, WSC)],
                mb.at[0], l0)
            for k in range(8):
                b = k % 2
                hl[k].wait()
                if k + 1 < 8:
                    if k >= 1:
                        for h in hs[k - 1]:
                            h.wait()
                    hl[k + 1] = pltpu.async_copy(
                        msg_hbm.at[c, pl.ds(g0 + (k + 1) * 128, 128),
                                   pl.ds(coff, WSC)],
                        mb.at[1 - b], lsems[1 - b])
                _sc_pows(mb.at[b], pb2.at[b], pb3.at[b], pb4.at[b], 128)
                idxrow = idx_v.at[k]
                hs[k] = [
                    pltpu.async_copy(mb.at[b], acc.at[0].at[idxrow],
                                     ssems[b], add=True),
                    pltpu.async_copy(pb2.at[b], acc.at[1].at[idxrow],
                                     ssems[b], add=True),
                    pltpu.async_copy(pb3.at[b], acc.at[2].at[idxrow],
                                     ssems[b], add=True),
                    pltpu.async_copy(pb4.at[b], acc.at[3].at[idxrow],
                                     ssems[b], add=True),
                ]
            for h in hs[6]:
                h.wait()
            for h in hs[7]:
                h.wait()
            return carry
        lax.fori_loop(0, NCH // 8, group, 0)

        plsc.subcore_barrier()
        for m in range(4):
            for z in range(5):
                r0 = t * 640 + z * 128
                pltpu.sync_copy(acc.at[m, pl.ds(r0, 128)],
                                out_hbm.at[m, c, pl.ds(r0, 128),
                                           pl.ds(coff, WSC)])
        plsc.subcore_barrier()


def _node_body(xs_ref, sums_ref, deg_ref, u_ref, w2a_ref, b2a_ref,
               w2b_ref, b2b_ref, gamma_ref, beta_ref, out_ref,
               h_all, acc, stats):
    phase = pl.program_id(0)
    j = pl.program_id(1)

    @pl.when(phase == 0)
    def _compute():
        s = sums_ref[...]
        degc = jnp.maximum(deg_ref[...][:, :1], 1.0)
        mean = jnp.concatenate([s[0, 0], s[0, 1]], axis=1) / degc
        m2 = jnp.concatenate([s[1, 0], s[1, 1]], axis=1) / degc
        m3 = jnp.concatenate([s[2, 0], s[2, 1]], axis=1) / degc
        m4 = jnp.concatenate([s[3, 0], s[3, 1]], axis=1) / degc
        var = _leaky(m2 - mean ** 2, 0.01)
        std = jnp.sqrt(var + 1e-06)
        cm3 = m3 - 3.0 * mean * m2 + 2.0 * mean ** 3
        cm4 = m4 - 4.0 * mean * m3 + 6.0 * (mean ** 2) * m2 - 3.0 * mean ** 4
        skew = jnp.nan_to_num(cm3 / (std ** 3), nan=0.0)
        kurt = jnp.nan_to_num(cm4 / (std ** 4), nan=0.0)
        u_exp = jnp.broadcast_to(u_ref[...], (NBLK, L))
        h_cat = jnp.concatenate([xs_ref[...], mean, std, skew, kurt, u_exp],
                                axis=1)
        t = _leaky(_dot_t(h_cat, w2a_ref[...]) + b2a_ref[...], 0.1)
        h = _dot_t(t, w2b_ref[...]) + b2b_ref[...]
        h_all[pl.ds(j * NBLK, NBLK), :] = h

        @pl.when(j == 0)
        def _init():
            acc[...] = jnp.zeros_like(acc)

        acc[0:1, :] += jnp.sum(h, axis=0, keepdims=True)

    @pl.when(jnp.logical_and(phase == 1, j == 0))
    def _stats():
        mu = acc[0:1, :] / float(N)
        d = h_all[...] - mu
        v = jnp.sum(d * d, axis=0, keepdims=True) / float(N)
        stats[0:1, :] = mu
        stats[1:2, :] = v

    @pl.when(phase == 1)
    def _apply():
        mu = stats[0:1, :]
        v = stats[1:2, :]
        h = h_all[pl.ds(j * NBLK, NBLK), :]
        out_ref[...] = (gamma_ref[...] * (h - mu) / jnp.sqrt(v + 1e-05)
                        + beta_ref[...])


def _node_out(x_s, sums, degb, u, W2a, b2a, W2b, b2b, gamma, beta):
    nsteps = N // NBLK
    return pl.pallas_call(
        _node_body,
        grid=(2, nsteps),
        in_specs=[
            pl.BlockSpec((NBLK, L), lambda p, j: (j, 0)),
            pl.BlockSpec((4, 2, NBLK, L), lambda p, j: (0, 0, j, 0)),

            pl.BlockSpec((NBLK, L), lambda p, j: (j, 0)),
            pl.BlockSpec((1, L), lambda p, j: (0, 0)),
            pl.BlockSpec((D2, D2), lambda p, j: (0, 0)),
            pl.BlockSpec((1, D2), lambda p, j: (0, 0)),
            pl.BlockSpec((L, D2), lambda p, j: (0, 0)),
            pl.BlockSpec((1, L), lambda p, j: (0, 0)),
            pl.BlockSpec((1, L), lambda p, j: (0, 0)),
            pl.BlockSpec((1, L), lambda p, j: (0, 0)),
        ],
        out_specs=pl.BlockSpec((NBLK, L), lambda p, j: (j, 0)),
        out_shape=jax.ShapeDtypeStruct((N, L), jnp.float32),
        scratch_shapes=[
            pltpu.VMEM((N, L), jnp.float32),
            pltpu.VMEM((1, L), jnp.float32),
            pltpu.VMEM((2, L), jnp.float32),
        ],
        interpret=_INTERPRET,
    )(x_s, sums, degb, u, W2a, b2a.reshape(1, D2), W2b,
      b2b.reshape(1, L), gamma.reshape(1, L), beta.reshape(1, L))


def kernel(x_s, x_t, edge_index, edge_attr, u, W1a, b1a, W1b, b1b,
           W2a, b2a, W2b, b2b, gamma, beta):
    src = edge_index[0]
    tgt = edge_index[1]
    xt_pad = jnp.zeros((ACC_ROWS, L), x_t.dtype).at[:N].set(x_t)
    tgt_pad = jnp.concatenate(
        [tgt, jnp.zeros((E_PAD - E,), jnp.int32)]).reshape(32, GCH, 128)
    xt_g = _sc_gather(xt_pad, tgt_pad)
    msg = _edge_msg(xt_g, edge_attr, W1a, b1a, W1b, b1b)
    src_pad = jnp.concatenate(
        [src, jnp.full((E_PAD - E,), DUMMY, jnp.int32)]).reshape(
            NTILES, EPT // 128, 128)
    sums = _sc_scatter(msg, src_pad)
    deg = jax.ops.segment_sum(jnp.ones((E,), jnp.float32), src,
                              num_segments=N)
    degb = jnp.broadcast_to(deg[:, None], (N, L))
    return _node_out(x_s, sums, degb, u, W2a, b2a, W2b, b2b, gamma, beta)


# speedup vs baseline: 1.0189x; 1.0189x over previous
import functools

import jax
import jax.numpy as jnp
from jax import lax
from jax.experimental import pallas as pl
from jax.experimental.pallas import tpu as pltpu
from jax.experimental.pallas import tpu_sc as plsc

L = 128
N = 10000
E = 320000
D1 = 2 * L
D2 = 4 * D1 + 2 * L

EBLK = 1024
E_PAD = 327680
NBLK = 1000

NTILES = 16
EPT = E_PAD // NTILES
NCH = 160
ACC_ROWS = 10240
DUMMY = N
W = 128

_INTERPRET = False


def _leaky(x, slope):
    return jnp.where(x >= 0, x, slope * x)


def _dot_t(a, w):
    return jax.lax.dot_general(a, w, (((1,), (1,)), ((), ())),
                               preferred_element_type=jnp.float32)


GPT = E_PAD // 32
GCH = GPT // 128


@functools.partial(
    pl.kernel,
    out_type=jax.ShapeDtypeStruct((E_PAD, L), jnp.float32),
    mesh=plsc.VectorSubcoreMesh(core_axis_name="c", subcore_axis_name="s"),
    scratch_types=[
        pltpu.VMEM_SHARED((ACC_ROWS, L), jnp.float32),
        pltpu.VMEM((GCH, 128), jnp.int32),
        pltpu.VMEM((128, L), jnp.float32),
        pltpu.VMEM((128, L), jnp.float32),
        pltpu.SemaphoreType.DMA,
        pltpu.SemaphoreType.DMA,
        pltpu.SemaphoreType.DMA,
        pltpu.SemaphoreType.DMA,
    ],
)
def _sc_gather(xt_hbm, tgt_hbm, out_hbm, xts, idx_v, r0b, r1b, g0, g1, w0, w1):
    c = lax.axis_index("c")
    s = lax.axis_index("s")
    w = s * 2 + c
    pltpu.sync_copy(xt_hbm.at[pl.ds(s * 640, 640)], xts.at[pl.ds(s * 640, 640)])
    pltpu.sync_copy(tgt_hbm.at[w], idx_v)
    plsc.subcore_barrier()

    rbs = (r0b, r1b)
    gsems = (g0, g1)
    wsems = (w0, w1)
    base = w * GPT

    def group(g, carry):
        hg = [None] * 8
        hw = [None] * 8
        row0 = base + g * (8 * 128)
        hg[0] = pltpu.async_copy(xts.at[idx_v.at[g * 8]], r0b, g0)
        for k in range(8):
            b = k % 2
            hg[k].wait()
            if k + 1 < 8:
                if k >= 1:
                    hw[k - 1].wait()
                hg[k + 1] = pltpu.async_copy(
                    xts.at[idx_v.at[g * 8 + k + 1]], rbs[1 - b],
                    gsems[1 - b])
            hw[k] = pltpu.async_copy(
                rbs[b], out_hbm.at[pl.ds(row0 + k * 128, 128)], wsems[b])
        hw[6].wait()
        hw[7].wait()
        return carry
    lax.fori_loop(0, GCH // 8, group, 0)


def _edge_body(xt_ref, ea_ref, w1a_ref, b1a_ref, w1b_ref, b1b_ref, msg_ref):
    x = jnp.concatenate([xt_ref[...], ea_ref[...]], axis=1)
    h = _dot_t(x, w1a_ref[...]) + b1a_ref[...]
    h = _leaky(h, 0.1)
    msg = _dot_t(h, w1b_ref[...]) + b1b_ref[...]
    msg_ref[0] = msg[:, :L]
    msg_ref[1] = msg[:, L:]


def _edge_msg(xt_g, edge_attr, W1a, b1a, W1b, b1b):
    nsteps = E_PAD // EBLK
    last = E // EBLK - (1 if E % EBLK == 0 else 0)

    def in_map(j):
        return (jnp.minimum(j, last), 0)

    return pl.pallas_call(
        _edge_body,
        grid=(nsteps,),
        in_specs=[
            pl.BlockSpec((EBLK, L), lambda j: (j, 0)),
            pl.BlockSpec((EBLK, L), in_map),
            pl.BlockSpec((D1, D1), lambda j: (0, 0)),
            pl.BlockSpec((1, D1), lambda j: (0, 0)),
            pl.BlockSpec((D1, D1), lambda j: (0, 0)),
            pl.BlockSpec((1, D1), lambda j: (0, 0)),
        ],
        out_specs=pl.BlockSpec((2, EBLK, L), lambda j: (0, j, 0)),
        out_shape=jax.ShapeDtypeStruct((2, E_PAD, L), jnp.float32),
        interpret=_INTERPRET,
    )(xt_g, edge_attr, W1a, b1a.reshape(1, D1), W1b, b1b.reshape(1, D1))


WSC = 32


WSC = 32


def _sc_pows(mb, pb2, pb3, pb4, nrows):
    @plsc.parallel_loop(0, nrows, unroll=4)
    def pw(r):
        for j in range(WSC // 16):
            v = mb[r, pl.ds(j * 16, 16)]
            v2 = v * v
            pb2[r, pl.ds(j * 16, 16)] = v2
            pb3[r, pl.ds(j * 16, 16)] = v2 * v
            pb4[r, pl.ds(j * 16, 16)] = v2 * v2


@functools.partial(
    pl.kernel,
    out_type=jax.ShapeDtypeStruct((4, 2, ACC_ROWS, L), jnp.float32),
    mesh=plsc.VectorSubcoreMesh(core_axis_name="c", subcore_axis_name="s"),
    compiler_params=pltpu.CompilerParams(use_tc_tiling_on_sc=False),
    scratch_types=[
        pltpu.VMEM_SHARED((4, ACC_ROWS, WSC), jnp.float32),
        pltpu.VMEM((8, 128), jnp.int32),
        pltpu.VMEM((2, 128, WSC), jnp.float32),
        pltpu.VMEM((2, 128, WSC), jnp.float32),
        pltpu.VMEM((2, 128, WSC), jnp.float32),
        pltpu.VMEM((2, 128, WSC), jnp.float32),
        pltpu.VMEM((128, WSC), jnp.float32),
        pltpu.SemaphoreType.DMA,
        pltpu.SemaphoreType.DMA,
        pltpu.SemaphoreType.DMA,
        pltpu.SemaphoreType.DMA,
    ],
)
def _sc_scatter(msg_hbm, idx_hbm, out_hbm, acc, idx_v, mb, pb2, pb3, pb4,
                zb, l0, l1, s0, s1):
    c = lax.axis_index("c")
    t = lax.axis_index("s")
    base = t * EPT
    lsems = (l0, l1)
    ssems = (s0, s1)

    zv = jnp.zeros((16,), jnp.float32)

    def _z(i, carry):
        for j in range(WSC // 16):
            zb[i, pl.ds(j * 16, 16)] = zv
        return carry
    lax.fori_loop(0, 128, _z, 0)

    for cg in range(4):
        coff = cg * WSC

        for m in range(4):
            for z in range(5):
                pltpu.sync_copy(zb,
                                acc.at[m, pl.ds(t * 640 + z * 128, 128)])
        plsc.subcore_barrier()

        def group(g, carry):
            pltpu.sync_copy(idx_hbm.at[t, pl.ds(g * 8, 8)], idx_v)
            g0 = base + g * (8 * 128)
            hl = [None] * 8
            hs = [None] * 8
            hl[0] = pltpu.async_copy(
                msg_hbm.at[c, pl.ds(g0, 128), pl.ds(coff, WSC)],
                mb.at[0], l0)
            for k in range(8):
                b = k % 2
                hl[k].wait()
                if k + 1 < 8:
                    if k >= 1:
                        for h in hs[k - 1]:
                            h.wait()
                    hl[k + 1] = pltpu.async_copy(
                        msg_hbm.at[c, pl.ds(g0 + (k + 1) * 128, 128),
                                   pl.ds(coff, WSC)],
                        mb.at[1 - b], lsems[1 - b])
                _sc_pows(mb.at[b], pb2.at[b], pb3.at[b], pb4.at[b], 128)
                idxrow = idx_v.at[k]
                hs[k] = [
                    pltpu.async_copy(mb.at[b], acc.at[0].at[idxrow],
                                     ssems[b], add=True),
                    pltpu.async_copy(pb2.at[b], acc.at[1].at[idxrow],
                                     ssems[b], add=True),
                    pltpu.async_copy(pb3.at[b], acc.at[2].at[idxrow],
                                     ssems[b], add=True),
                    pltpu.async_copy(pb4.at[b], acc.at[3].at[idxrow],
                                     ssems[b], add=True),
                ]
            for h in hs[6]:
                h.wait()
            for h in hs[7]:
                h.wait()
            return carry
        lax.fori_loop(0, NCH // 8, group, 0)

        plsc.subcore_barrier()
        for m in range(4):
            for z in range(5):
                r0 = t * 640 + z * 128
                pltpu.sync_copy(acc.at[m, pl.ds(r0, 128)],
                                out_hbm.at[m, c, pl.ds(r0, 128),
                                           pl.ds(coff, WSC)])
        plsc.subcore_barrier()


def _node_body(xs_ref, sums_ref, deg_ref, u_ref, w2a_ref, b2a_ref,
               w2b_ref, b2b_ref, gamma_ref, beta_ref, out_ref,
               h_all, acc, stats):
    phase = pl.program_id(0)
    j = pl.program_id(1)

    @pl.when(phase == 0)
    def _compute():
        s = sums_ref[...]
        degc = jnp.maximum(deg_ref[...][:, :1], 1.0)
        mean = jnp.concatenate([s[0, 0], s[0, 1]], axis=1) / degc
        m2 = jnp.concatenate([s[1, 0], s[1, 1]], axis=1) / degc
        m3 = jnp.concatenate([s[2, 0], s[2, 1]], axis=1) / degc
        m4 = jnp.concatenate([s[3, 0], s[3, 1]], axis=1) / degc
        var = _leaky(m2 - mean ** 2, 0.01)
        std = jnp.sqrt(var + 1e-06)
        cm3 = m3 - 3.0 * mean * m2 + 2.0 * mean ** 3
        cm4 = m4 - 4.0 * mean * m3 + 6.0 * (mean ** 2) * m2 - 3.0 * mean ** 4
        skew = jnp.nan_to_num(cm3 / (std ** 3), nan=0.0)
        kurt = jnp.nan_to_num(cm4 / (std ** 4), nan=0.0)
        u_exp = jnp.broadcast_to(u_ref[...], (NBLK, L))
        h_cat = jnp.concatenate([xs_ref[...], mean, std, skew, kurt, u_exp],
                                axis=1)
        t = _leaky(_dot_t(h_cat, w2a_ref[...]) + b2a_ref[...], 0.1)
        h = _dot_t(t, w2b_ref[...]) + b2b_ref[...]
        h_all[pl.ds(j * NBLK, NBLK), :] = h

        @pl.when(j == 0)
        def _init():
            acc[...] = jnp.zeros_like(acc)

        acc[0:1, :] += jnp.sum(h, axis=0, keepdims=True)

    @pl.when(jnp.logical_and(phase == 1, j == 0))
    def _stats():
        mu = acc[0:1, :] / float(N)
        d = h_all[...] - mu
        v = jnp.sum(d * d, axis=0, keepdims=True) / float(N)
        stats[0:1, :] = mu
        stats[1:2, :] = v

    @pl.when(phase == 1)
    def _apply():
        mu = stats[0:1, :]
        v = stats[1:2, :]
        h = h_all[pl.ds(j * NBLK, NBLK), :]
        out_ref[...] = (gamma_ref[...] * (h - mu) / jnp.sqrt(v + 1e-05)
                        + beta_ref[...])


def _node_out(x_s, sums, degb, u, W2a, b2a, W2b, b2b, gamma, beta):
    nsteps = N // NBLK
    return pl.pallas_call(
        _node_body,
        grid=(2, nsteps),
        in_specs=[
            pl.BlockSpec((NBLK, L), lambda p, j: (j, 0)),
            pl.BlockSpec((4, 2, NBLK, L), lambda p, j: (0, 0, j, 0)),

            pl.BlockSpec((NBLK, L), lambda p, j: (j, 0)),
            pl.BlockSpec((1, L), lambda p, j: (0, 0)),
            pl.BlockSpec((D2, D2), lambda p, j: (0, 0)),
            pl.BlockSpec((1, D2), lambda p, j: (0, 0)),
            pl.BlockSpec((L, D2), lambda p, j: (0, 0)),
            pl.BlockSpec((1, L), lambda p, j: (0, 0)),
            pl.BlockSpec((1, L), lambda p, j: (0, 0)),
            pl.BlockSpec((1, L), lambda p, j: (0, 0)),
        ],
        out_specs=pl.BlockSpec((NBLK, L), lambda p, j: (j, 0)),
        out_shape=jax.ShapeDtypeStruct((N, L), jnp.float32),
        scratch_shapes=[
            pltpu.VMEM((N, L), jnp.float32),
            pltpu.VMEM((1, L), jnp.float32),
            pltpu.VMEM((2, L), jnp.float32),
        ],
        interpret=_INTERPRET,
    )(x_s, sums, degb, u, W2a, b2a.reshape(1, D2), W2b,
      b2b.reshape(1, L), gamma.reshape(1, L), beta.reshape(1, L))


def kernel(x_s, x_t, edge_index, edge_attr, u, W1a, b1a, W1b, b1b,
           W2a, b2a, W2b, b2b, gamma, beta):
    src = edge_index[0]
    tgt = edge_index[1]
    xt_pad = jnp.zeros((ACC_ROWS, L), x_t.dtype).at[:N].set(x_t)
    tgt_pad = jnp.concatenate(
        [tgt, jnp.zeros((E_PAD - E,), jnp.int32)]).reshape(32, GCH, 128)
    xt_g = _sc_gather(xt_pad, tgt_pad)
    msg = _edge_msg(xt_g, edge_attr, W1a, b1a, W1b, b1b)
    src_pad = jnp.concatenate(
        [src, jnp.full((E_PAD - E,), DUMMY, jnp.int32)]).reshape(
            NTILES, EPT // 128, 128)
    sums = _sc_scatter(msg, src_pad)
    deg = jax.ops.segment_sum(jnp.ones((E,), jnp.float32), src,
                              num_segments=N)
    degb = jnp.broadcast_to(deg[:, None], (N, L))
    return _node_out(x_s, sums, degb, u, W2a, b2a, W2b, b2b, gamma, beta)
